# Initial kernel scaffold; baseline (speedup 1.0000x reference)
import jax
import jax.numpy as jnp
from jax.experimental import pallas as pl


def kernel(tc, embedding):
    raise NotImplementedError("write your pallas kernel here")


# speedup vs baseline: 3.6416x; 3.6416x over previous
import functools

import jax
import jax.numpy as jnp
from jax import lax
from jax.experimental import pallas as pl
from jax.experimental.pallas import tpu as pltpu
from jax.experimental.pallas import tpu_sc as plsc

NC = 2
NS = 16
NW = NC * NS

D = 64
IDX_W = 128
GROUP = 8


def _tile_body(nrows_per_w, idx_hbm, table_hbm, out_hbm, idx_v, rows_v, gsem, ssem):
    wid = lax.axis_index("s") * NC + lax.axis_index("c")
    row0 = wid * nrows_per_w
    ngroups = nrows_per_w // GROUP

    def group_step(g, carry):
        grow = row0 + g * GROUP
        pltpu.sync_copy(idx_hbm.at[pl.ds(grow, GROUP)], idx_v)
        for b in range(GROUP):
            @pl.when(g > 0)
            def _wait_prev():
                pltpu.make_async_copy(
                    rows_v.at[b], out_hbm.at[pl.ds(0, IDX_W)], ssem.at[b]
                ).wait()

            pltpu.async_copy(table_hbm.at[idx_v.at[b]], rows_v.at[b], gsem.at[b])
        for b in range(GROUP):
            pltpu.make_async_copy(
                table_hbm.at[idx_v.at[b]], rows_v.at[b], gsem.at[b]
            ).wait()
            pltpu.async_copy(
                rows_v.at[b], out_hbm.at[pl.ds((grow + b) * IDX_W, IDX_W)], ssem.at[b]
            )
        return carry

    lax.fori_loop(0, ngroups, group_step, 0)
    for b in range(GROUP):
        pltpu.make_async_copy(
            rows_v.at[b], out_hbm.at[pl.ds(0, IDX_W)], ssem.at[b]
        ).wait()


def kernel(tc, embedding):
    bsz, hist = tc.shape
    total = bsz * hist
    assert total % (NW * IDX_W) == 0
    nrows = total // IDX_W
    nrows_per_w = nrows // NW
    assert nrows_per_w % GROUP == 0

    idx = tc.reshape(nrows, IDX_W).astype(jnp.int32)
    mesh = plsc.VectorSubcoreMesh(
        core_axis_name="c", subcore_axis_name="s", num_cores=NC, num_subcores=NS
    )
    run = pl.kernel(
        functools.partial(_tile_body, nrows_per_w),
        out_type=jax.ShapeDtypeStruct((total, D), jnp.float32),
        mesh=mesh,
        scratch_types=[
            pltpu.VMEM((GROUP, IDX_W), jnp.int32),
            pltpu.VMEM((GROUP, IDX_W, D), jnp.float32),
            pltpu.SemaphoreType.DMA((GROUP,)),
            pltpu.SemaphoreType.DMA((GROUP,)),
        ],
        compiler_params=pltpu.CompilerParams(use_tc_tiling_on_sc=False),
    )
    out = run(idx, embedding)
    return out.reshape(bsz, hist, D)


# speedup vs baseline: 5.8239x; 1.5993x over previous
import functools

import jax
import jax.numpy as jnp
from jax import lax
from jax.experimental import pallas as pl
from jax.experimental.pallas import tpu as pltpu
from jax.experimental.pallas import tpu_sc as plsc

NC = 2
NS = 16
NW = NC * NS

D = 64
TW = 128
IDX_W = 128
GROUP = 8
NBUF = 4
L = 16


def _tile_body(nrows_per_w, nrowstab, idx_hbm, table_hbm, out_hbm,
               idx_v, tab_v, obuf, ssem):
    wid = lax.axis_index("s") * NC + lax.axis_index("c")
    row0 = wid * nrows_per_w
    ngroups = nrows_per_w // GROUP

    pltpu.sync_copy(table_hbm, tab_v)

    def group_step(g, carry):
        grow = row0 + g * GROUP
        pltpu.sync_copy(idx_hbm.at[pl.ds(grow, GROUP)], idx_v)
        for j in range(GROUP):
            b = j % NBUF

            def _wait_prev_store():
                pltpu.make_async_copy(
                    obuf.at[b], out_hbm.at[pl.ds(0, IDX_W)], ssem.at[b]
                ).wait()

            if j >= NBUF:
                _wait_prev_store()
            else:
                pl.when(g > 0)(_wait_prev_store)

            def row16_step(k, carry2):
                iv = idx_v[j, pl.ds(k * L, L)]
                r0 = k * L
                for l in range(L):
                    i = iv[l]
                    for c in range(D // L):
                        obuf[b, r0 + l, pl.ds(c * L, L)] = tab_v[i, pl.ds(c * L, L)]
                return carry2

            lax.fori_loop(0, IDX_W // L, row16_step, 0)
            pltpu.async_copy(
                obuf.at[b], out_hbm.at[pl.ds((grow + j) * IDX_W, IDX_W)],
                ssem.at[b],
            )
        return carry

    lax.fori_loop(0, ngroups, group_step, 0)
    for b in range(NBUF):
        pltpu.make_async_copy(
            obuf.at[b], out_hbm.at[pl.ds(0, IDX_W)], ssem.at[b]
        ).wait()


def kernel(tc, embedding):
    bsz, hist = tc.shape
    total = bsz * hist
    assert total % (NW * IDX_W) == 0
    nrows = total // IDX_W
    nrows_per_w = nrows // NW
    assert nrows_per_w % GROUP == 0

    idx = tc.reshape(nrows, IDX_W).astype(jnp.int32)
    table = jnp.pad(embedding, ((0, 0), (0, TW - embedding.shape[1])))
    mesh = plsc.VectorSubcoreMesh(
        core_axis_name="c", subcore_axis_name="s", num_cores=NC, num_subcores=NS
    )
    run = pl.kernel(
        functools.partial(_tile_body, nrows_per_w, table.shape[0]),
        out_type=jax.ShapeDtypeStruct((total, D), jnp.float32),
        mesh=mesh,
        scratch_types=[
            pltpu.VMEM((GROUP, IDX_W), jnp.int32),
            pltpu.VMEM(table.shape, jnp.float32),
            pltpu.VMEM((NBUF, IDX_W, D), jnp.float32),
            pltpu.SemaphoreType.DMA((NBUF,)),
        ],
    )
    out = run(idx, table)
    return out.reshape(bsz, hist, D)
